# baseline (speedup 1.0000x reference)
import functools

import jax
import jax.numpy as jnp
from jax import lax
from jax.experimental import pallas as pl
from jax.experimental.pallas import tpu as pltpu
from jax.experimental.pallas import tpu_sc as plsc

ROWS = 128
COLS = 32768
Q_IDX = int(COLS * (1 - 0.1))
TARGET = COLS - Q_IDX
NB1 = 256
NB2 = 2048
NB3 = 4096
L = 16


def _zero_upto(cnt_ref, sum_ref, nwords):
    zi = jnp.zeros((L,), jnp.int32)
    zf = jnp.zeros((L,), jnp.float32)

    def body(j, c):
        cnt_ref[pl.ds(j * L, L)] = zi
        sum_ref[pl.ds(j * L, L)] = zf
        return c

    lax.fori_loop(0, (nwords + L - 1) // L, body, jnp.int32(0))


def _walk_lanes(cnt_ref, sum_ref, start_b, t):
    def cond(c):
        return (c[1] == 0) & (c[0] >= 0)

    def body(c):
        b, found, b_sel, ab_cnt, ab_sum, acc_cnt, acc_sum = c
        cnt = jnp.sum(cnt_ref[pl.ds(b * L, L)])
        sm = jnp.sum(sum_ref[pl.ds(b * L, L)])
        hit = jnp.where(acc_cnt + cnt >= t, jnp.int32(1), jnp.int32(0))
        b_sel = jnp.where(hit == 1, b, b_sel)
        ab_cnt = jnp.where(hit == 1, acc_cnt, ab_cnt)
        ab_sum = jnp.where(hit == 1, acc_sum, ab_sum)
        acc_cnt = jnp.where(hit == 1, acc_cnt, acc_cnt + cnt)
        acc_sum = jnp.where(hit == 1, acc_sum, acc_sum + sm)
        return (b - jnp.int32(1), found | hit, b_sel, ab_cnt, ab_sum,
                acc_cnt, acc_sum)

    init = (start_b, jnp.int32(0), jnp.int32(0), jnp.int32(0),
            jnp.float32(0.0), jnp.int32(0), jnp.float32(0.0))
    out = lax.while_loop(cond, body, init)
    return out[2], out[3], out[4]


def _walk(cnt_ref, sum_ref, start_blk, t):
    iota = lax.iota(jnp.int32, L)

    def cond(carry):
        return (carry[1] == 0) & (carry[0] >= 0)

    def body(carry):
        k, found, b_sel, ab_cnt, ab_sum, acc_cnt, acc_sum = carry
        base = k * L
        rc = lax.rev(cnt_ref[pl.ds(base, L)], (0,))
        rs = lax.rev(sum_ref[pl.ds(base, L)], (0,))
        c = lax.cumsum(rc, axis=0)
        blk_cnt = jnp.max(c)
        blk_sum = jnp.sum(rs)
        hit = jnp.where(acc_cnt + blk_cnt >= t, jnp.int32(1), jnp.int32(0))
        ge = (acc_cnt + c) >= t
        i_star = jnp.min(jnp.where(ge, iota, jnp.int32(L)))
        within_cnt = jnp.sum(jnp.where(iota < i_star, rc, jnp.int32(0)))
        within_sum = jnp.sum(jnp.where(iota < i_star, rs, jnp.float32(0.0)))
        b_new = base + jnp.int32(L - 1) - i_star
        b_sel = jnp.where(hit == 1, b_new, b_sel)
        ab_cnt = jnp.where(hit == 1, acc_cnt + within_cnt, ab_cnt)
        ab_sum = jnp.where(hit == 1, acc_sum + within_sum, ab_sum)
        acc_cnt = jnp.where(hit == 1, acc_cnt, acc_cnt + blk_cnt)
        acc_sum = jnp.where(hit == 1, acc_sum, acc_sum + blk_sum)
        return (k - jnp.int32(1), found | hit, b_sel, ab_cnt, ab_sum,
                acc_cnt, acc_sum)

    init = (start_blk, jnp.int32(0), jnp.int32(0), jnp.int32(0),
            jnp.float32(0.0), jnp.int32(0), jnp.float32(0.0))
    out = lax.while_loop(cond, body, init)
    return out[2], out[3], out[4]


def _pass1(row_ref, cnt_ref, sum_ref):
    ones = jnp.full((L,), 1, jnp.int32)
    iota = lax.iota(jnp.int32, L)
    UN = 8

    def body(j, kmax):
        for u in range(UN):
            v = row_ref[pl.ds((j * UN + u) * L, L)]
            a = jnp.abs(v)
            bits = lax.bitcast_convert_type(a, jnp.int32)
            idx = (lax.shift_right_logical(bits, 19) & jnp.int32(0xFF0)) | iota
            plsc.addupdate_scatter(cnt_ref, [idx], ones)
            plsc.addupdate_scatter(sum_ref, [idx], a)
            kmax = jnp.maximum(kmax, bits)
        return kmax

    kmax = lax.fori_loop(0, COLS // L // UN, body, jnp.zeros((L,), jnp.int32))
    return lax.shift_right_logical(jnp.max(kmax), 23)


def _pass23(row_ref, cnt_ref, sum_ref, pre_shift, prefix, key_shift, key_msk):
    ones = jnp.full((L,), 1, jnp.int32)
    UN = 4

    def body(j, kmax):
        for u in range(UN):
            v = row_ref[pl.ds((j * UN + u) * L, L)]
            a = jnp.abs(v)
            bits = lax.bitcast_convert_type(a, jnp.int32)
            m = lax.shift_right_logical(bits, pre_shift) == prefix
            key = lax.shift_right_logical(bits, key_shift) & key_msk
            plsc.addupdate_scatter(cnt_ref, [key], ones, mask=m)
            plsc.addupdate_scatter(sum_ref, [key], a, mask=m)
            kmax = jnp.maximum(kmax, jnp.where(m, key, jnp.int32(0)))
        return kmax

    kmax = lax.fori_loop(0, COLS // L // UN, body, jnp.zeros((L,), jnp.int32))
    return jnp.max(kmax)


def _select_row(row_ref, h1c, h1s, h2c, h2s, h3c, h3s):
    kmax1 = _pass1(row_ref, h1c, h1s)
    b1, ac1, as1 = _walk_lanes(h1c, h1s, kmax1, jnp.int32(TARGET))
    t2 = jnp.int32(TARGET) - ac1
    kmax2 = _pass23(row_ref, h2c, h2s, 23, b1, 12, jnp.int32(NB2 - 1))
    b2, ac2, as2 = _walk(h2c, h2s, lax.shift_right_logical(kmax2, 4), t2)
    t3 = t2 - ac2
    pre3 = lax.shift_left(b1, 11) | b2
    kmax3 = _pass23(row_ref, h3c, h3s, 12, pre3, 0, jnp.int32(NB3 - 1))
    _, _, as3 = _walk(h3c, h3s, lax.shift_right_logical(kmax3, 4), t3)
    _zero_upto(h1c, h1s, (kmax1 + 1) * L)
    _zero_upto(h2c, h2s, kmax2 + 1)
    _zero_upto(h3c, h3s, kmax3 + 1)
    return as1 + as2 + as3


def _make_selector():
    info = plsc.get_sparse_core_info()
    nw = info.num_cores * info.num_subcores
    rows_per_w = ROWS // nw
    mesh = plsc.VectorSubcoreMesh(core_axis_name="c", subcore_axis_name="s")

    @functools.partial(
        pl.kernel,
        mesh=mesh,
        compiler_params=pltpu.CompilerParams(needs_layout_passes=False),
        out_type=jax.ShapeDtypeStruct((nw, L), jnp.float32),
        scratch_types=[
            pltpu.VMEM((COLS,), jnp.float32),
            pltpu.VMEM((COLS,), jnp.float32),
            pltpu.VMEM((NB1 * L,), jnp.int32),
            pltpu.VMEM((NB1 * L,), jnp.float32),
            pltpu.VMEM((NB2,), jnp.int32),
            pltpu.VMEM((NB2,), jnp.float32),
            pltpu.VMEM((NB3,), jnp.int32),
            pltpu.VMEM((NB3,), jnp.float32),
            pltpu.VMEM((L,), jnp.float32),
            pltpu.SemaphoreType.DMA,
            pltpu.SemaphoreType.DMA,
        ],
    )
    def sel(masks_hbm, out_hbm, row_a, row_b, h1c, h1s, h2c, h2s, h3c, h3s,
            stage, sem_a, sem_b):
        wid = lax.axis_index("s") * info.num_cores + lax.axis_index("c")
        row0 = wid * rows_per_w
        iota = lax.iota(jnp.int32, L)
        bufs = (row_a, row_b)
        sems = (sem_a, sem_b)

        handles = {}
        for r in range(2):
            handles[r] = pltpu.async_copy(masks_hbm.at[row0 + r], bufs[r],
                                          sems[r])
        _zero_upto(h1c, h1s, NB1 * L)
        _zero_upto(h2c, h2s, NB2)
        _zero_upto(h3c, h3s, NB3)
        acc = jnp.zeros((L,), jnp.float32)
        for r in range(rows_per_w):
            handles[r].wait()
            ans = _select_row(bufs[r % 2], h1c, h1s, h2c, h2s, h3c, h3s)
            acc = jnp.where(iota == r, ans, acc)
            if r + 2 < rows_per_w:
                handles[r + 2] = pltpu.async_copy(
                    masks_hbm.at[row0 + r + 2], bufs[r % 2], sems[r % 2])
        stage[...] = acc
        pltpu.sync_copy(stage, out_hbm.at[wid])

    return sel


_selector = _make_selector()


def kernel(outputs_support, outputs_delete, targets, masks):
    parts = _selector(masks)
    return outputs_support[0] + 0.01 * jnp.sum(parts)


# speedup vs baseline: 2.6347x; 2.6347x over previous
import functools

import jax
import jax.numpy as jnp
from jax import lax
from jax.experimental import pallas as pl
from jax.experimental.pallas import tpu as pltpu
from jax.experimental.pallas import tpu_sc as plsc

ROWS = 128
COLS = 32768
Q_IDX = int(COLS * (1 - 0.1))
TARGET = COLS - Q_IDX
NB1 = 256
NB2 = 2048
NB3 = 4096
L = 16


def _zero_upto(cnt_ref, sum_ref, nwords):
    zi = jnp.zeros((L,), jnp.int32)
    zf = jnp.zeros((L,), jnp.float32)

    @plsc.parallel_loop(0, (nwords + L - 1) // L, unroll=4)
    def _(j):
        cnt_ref[pl.ds(j * L, L)] = zi
        sum_ref[pl.ds(j * L, L)] = zf


def _walk_lanes(cnt_ref, sum_ref, start_b, t):
    def cond(c):
        return (c[1] == 0) & (c[0] >= 0)

    def body(c):
        b, found, b_sel, ab_cnt, ab_sum, acc_cnt, acc_sum = c
        cnt = jnp.sum(cnt_ref[pl.ds(b * L, L)])
        sm = jnp.sum(sum_ref[pl.ds(b * L, L)])
        hit = jnp.where(acc_cnt + cnt >= t, jnp.int32(1), jnp.int32(0))
        b_sel = jnp.where(hit == 1, b, b_sel)
        ab_cnt = jnp.where(hit == 1, acc_cnt, ab_cnt)
        ab_sum = jnp.where(hit == 1, acc_sum, ab_sum)
        acc_cnt = jnp.where(hit == 1, acc_cnt, acc_cnt + cnt)
        acc_sum = jnp.where(hit == 1, acc_sum, acc_sum + sm)
        return (b - jnp.int32(1), found | hit, b_sel, ab_cnt, ab_sum,
                acc_cnt, acc_sum)

    init = (start_b, jnp.int32(0), jnp.int32(0), jnp.int32(0),
            jnp.float32(0.0), jnp.int32(0), jnp.float32(0.0))
    out = lax.while_loop(cond, body, init)
    return out[2], out[3], out[4]


def _walk(cnt_ref, sum_ref, start_blk, t):
    iota = lax.iota(jnp.int32, L)

    def cond(carry):
        return (carry[1] == 0) & (carry[0] >= 0)

    def body(carry):
        k, found, b_sel, ab_cnt, ab_sum, acc_cnt, acc_sum = carry
        base = k * L
        rc = lax.rev(cnt_ref[pl.ds(base, L)], (0,))
        rs = lax.rev(sum_ref[pl.ds(base, L)], (0,))
        c = lax.cumsum(rc, axis=0)
        blk_cnt = jnp.max(c)
        blk_sum = jnp.sum(rs)
        hit = jnp.where(acc_cnt + blk_cnt >= t, jnp.int32(1), jnp.int32(0))
        ge = (acc_cnt + c) >= t
        i_star = jnp.min(jnp.where(ge, iota, jnp.int32(L)))
        within_cnt = jnp.sum(jnp.where(iota < i_star, rc, jnp.int32(0)))
        within_sum = jnp.sum(jnp.where(iota < i_star, rs, jnp.float32(0.0)))
        b_new = base + jnp.int32(L - 1) - i_star
        b_sel = jnp.where(hit == 1, b_new, b_sel)
        ab_cnt = jnp.where(hit == 1, acc_cnt + within_cnt, ab_cnt)
        ab_sum = jnp.where(hit == 1, acc_sum + within_sum, ab_sum)
        acc_cnt = jnp.where(hit == 1, acc_cnt, acc_cnt + blk_cnt)
        acc_sum = jnp.where(hit == 1, acc_sum, acc_sum + blk_sum)
        return (k - jnp.int32(1), found | hit, b_sel, ab_cnt, ab_sum,
                acc_cnt, acc_sum)

    init = (start_blk, jnp.int32(0), jnp.int32(0), jnp.int32(0),
            jnp.float32(0.0), jnp.int32(0), jnp.float32(0.0))
    out = lax.while_loop(cond, body, init)
    return out[2], out[3], out[4]


def _pass1(row_ref, cnt_ref, sum_ref):
    ones = jnp.full((L,), 1, jnp.int32)
    iota = lax.iota(jnp.int32, L)

    @plsc.parallel_loop(0, COLS // L, unroll=8,
                        carry=jnp.zeros((L,), jnp.int32))
    def kmax(j, km):
        v = row_ref[pl.ds(j * L, L)]
        a = jnp.abs(v)
        bits = lax.bitcast_convert_type(a, jnp.int32)
        idx = (lax.shift_right_logical(bits, 19) & jnp.int32(0xFF0)) | iota
        plsc.addupdate_scatter(cnt_ref, [idx], ones)
        plsc.addupdate_scatter(sum_ref, [idx], a)
        return jnp.maximum(km, bits)

    return lax.shift_right_logical(jnp.max(kmax), 23)


def _pass23(row_ref, cnt_ref, sum_ref, pre_shift, prefix, key_shift, key_msk):
    ones = jnp.full((L,), 1, jnp.int32)

    @plsc.parallel_loop(0, COLS // L, unroll=8,
                        carry=jnp.zeros((L,), jnp.int32))
    def kmax(j, km):
        v = row_ref[pl.ds(j * L, L)]
        a = jnp.abs(v)
        bits = lax.bitcast_convert_type(a, jnp.int32)
        m = lax.shift_right_logical(bits, pre_shift) == prefix
        key = lax.shift_right_logical(bits, key_shift) & key_msk
        plsc.addupdate_scatter(cnt_ref, [key], ones, mask=m)
        plsc.addupdate_scatter(sum_ref, [key], a, mask=m)
        return jnp.maximum(km, jnp.where(m, key, jnp.int32(0)))

    return jnp.max(kmax)


def _select_row(row_ref, h1c, h1s, h2c, h2s, h3c, h3s):
    kmax1 = _pass1(row_ref, h1c, h1s)
    b1, ac1, as1 = _walk_lanes(h1c, h1s, kmax1, jnp.int32(TARGET))
    t2 = jnp.int32(TARGET) - ac1
    kmax2 = _pass23(row_ref, h2c, h2s, 23, b1, 12, jnp.int32(NB2 - 1))
    b2, ac2, as2 = _walk(h2c, h2s, lax.shift_right_logical(kmax2, 4), t2)
    t3 = t2 - ac2
    pre3 = lax.shift_left(b1, 11) | b2
    kmax3 = _pass23(row_ref, h3c, h3s, 12, pre3, 0, jnp.int32(NB3 - 1))
    _, _, as3 = _walk(h3c, h3s, lax.shift_right_logical(kmax3, 4), t3)
    _zero_upto(h1c, h1s, (kmax1 + 1) * L)
    _zero_upto(h2c, h2s, kmax2 + 1)
    _zero_upto(h3c, h3s, kmax3 + 1)
    return as1 + as2 + as3


def _make_selector():
    info = plsc.get_sparse_core_info()
    nw = info.num_cores * info.num_subcores
    rows_per_w = ROWS // nw
    mesh = plsc.VectorSubcoreMesh(core_axis_name="c", subcore_axis_name="s")

    @functools.partial(
        pl.kernel,
        mesh=mesh,
        compiler_params=pltpu.CompilerParams(needs_layout_passes=False),
        out_type=jax.ShapeDtypeStruct((nw, L), jnp.float32),
        scratch_types=[
            pltpu.VMEM((COLS,), jnp.float32),
            pltpu.VMEM((COLS,), jnp.float32),
            pltpu.VMEM((NB1 * L,), jnp.int32),
            pltpu.VMEM((NB1 * L,), jnp.float32),
            pltpu.VMEM((NB2,), jnp.int32),
            pltpu.VMEM((NB2,), jnp.float32),
            pltpu.VMEM((NB3,), jnp.int32),
            pltpu.VMEM((NB3,), jnp.float32),
            pltpu.VMEM((L,), jnp.float32),
            pltpu.SemaphoreType.DMA,
            pltpu.SemaphoreType.DMA,
        ],
    )
    def sel(masks_hbm, out_hbm, row_a, row_b, h1c, h1s, h2c, h2s, h3c, h3s,
            stage, sem_a, sem_b):
        wid = lax.axis_index("s") * info.num_cores + lax.axis_index("c")
        row0 = wid * rows_per_w
        iota = lax.iota(jnp.int32, L)
        bufs = (row_a, row_b)
        sems = (sem_a, sem_b)

        handles = {}
        for r in range(2):
            handles[r] = pltpu.async_copy(masks_hbm.at[row0 + r], bufs[r],
                                          sems[r])
        _zero_upto(h1c, h1s, NB1 * L)
        _zero_upto(h2c, h2s, NB2)
        _zero_upto(h3c, h3s, NB3)
        acc = jnp.zeros((L,), jnp.float32)
        for r in range(rows_per_w):
            handles[r].wait()
            ans = _select_row(bufs[r % 2], h1c, h1s, h2c, h2s, h3c, h3s)
            acc = jnp.where(iota == r, ans, acc)
            if r + 2 < rows_per_w:
                handles[r + 2] = pltpu.async_copy(
                    masks_hbm.at[row0 + r + 2], bufs[r % 2], sems[r % 2])
        stage[...] = acc
        pltpu.sync_copy(stage, out_hbm.at[wid])

    return sel


_selector = _make_selector()


def kernel(outputs_support, outputs_delete, targets, masks):
    parts = _selector(masks)
    return outputs_support[0] + 0.01 * jnp.sum(parts)


# speedup vs baseline: 2.9111x; 1.1049x over previous
import functools

import jax
import jax.numpy as jnp
from jax import lax
from jax.experimental import pallas as pl
from jax.experimental.pallas import tpu as pltpu
from jax.experimental.pallas import tpu_sc as plsc

ROWS = 128
COLS = 32768
Q_IDX = int(COLS * (1 - 0.1))
TARGET = COLS - Q_IDX
NB1 = 256
NB2 = 2048
NB3 = 4096
L = 16


def _zero_cnt(cnt_ref, nwords):
    zi = jnp.zeros((L,), jnp.int32)

    @plsc.parallel_loop(0, (nwords + L - 1) // L, unroll=4)
    def _(j):
        cnt_ref[pl.ds(j * L, L)] = zi


def _walk_lanes(cnt_ref, start_b, t):
    def cond(c):
        return (c[1] == 0) & (c[0] >= 0)

    def body(c):
        b, found, b_sel, ab_cnt, acc_cnt = c
        cnt = jnp.sum(cnt_ref[pl.ds(b * L, L)])
        hit = jnp.where(acc_cnt + cnt >= t, jnp.int32(1), jnp.int32(0))
        b_sel = jnp.where(hit == 1, b, b_sel)
        ab_cnt = jnp.where(hit == 1, acc_cnt, ab_cnt)
        acc_cnt = jnp.where(hit == 1, acc_cnt, acc_cnt + cnt)
        return (b - jnp.int32(1), found | hit, b_sel, ab_cnt, acc_cnt)

    init = (start_b, jnp.int32(0), jnp.int32(0), jnp.int32(0), jnp.int32(0))
    out = lax.while_loop(cond, body, init)
    return out[2], out[3]


def _walk(cnt_ref, start_blk, t):
    iota = lax.iota(jnp.int32, L)

    def cond(carry):
        return (carry[1] == 0) & (carry[0] >= 0)

    def body(carry):
        k, found, b_sel, ab_cnt, sel_cnt, acc_cnt = carry
        base = k * L
        rc = lax.rev(cnt_ref[pl.ds(base, L)], (0,))
        c = lax.cumsum(rc, axis=0)
        blk_cnt = jnp.max(c)
        hit = jnp.where(acc_cnt + blk_cnt >= t, jnp.int32(1), jnp.int32(0))
        ge = (acc_cnt + c) >= t
        i_star = jnp.min(jnp.where(ge, iota, jnp.int32(L)))
        within_cnt = jnp.sum(jnp.where(iota < i_star, rc, jnp.int32(0)))
        here_cnt = jnp.sum(jnp.where(iota == i_star, rc, jnp.int32(0)))
        b_new = base + jnp.int32(L - 1) - i_star
        b_sel = jnp.where(hit == 1, b_new, b_sel)
        ab_cnt = jnp.where(hit == 1, acc_cnt + within_cnt, ab_cnt)
        sel_cnt = jnp.where(hit == 1, here_cnt, sel_cnt)
        acc_cnt = jnp.where(hit == 1, acc_cnt, acc_cnt + blk_cnt)
        return (k - jnp.int32(1), found | hit, b_sel, ab_cnt, sel_cnt,
                acc_cnt)

    init = (start_blk, jnp.int32(0), jnp.int32(0), jnp.int32(0),
            jnp.int32(0), jnp.int32(0))
    out = lax.while_loop(cond, body, init)
    return out[2], out[3], out[4]


def _pass1(row_ref, cnt_ref):
    ones = jnp.full((L,), 1, jnp.int32)
    iota = lax.iota(jnp.int32, L)

    @plsc.parallel_loop(0, COLS // L, unroll=8,
                        carry=jnp.zeros((L,), jnp.int32))
    def kmax(j, km):
        v = row_ref[pl.ds(j * L, L)]
        bits = lax.bitcast_convert_type(v, jnp.int32) & jnp.int32(0x7FFFFFFF)
        idx = (lax.shift_right_logical(bits, 19) & jnp.int32(0xFF0)) | iota
        plsc.addupdate_scatter(cnt_ref, [idx], ones)
        return jnp.maximum(km, bits)

    return jnp.max(kmax)


def _pass2(row_ref, cnt_ref, b1):
    ones = jnp.full((L,), 1, jnp.int32)

    @plsc.parallel_loop(0, COLS // L, unroll=8,
                        carry=jnp.zeros((L,), jnp.int32))
    def kmax(j, km):
        v = row_ref[pl.ds(j * L, L)]
        bits = lax.bitcast_convert_type(v, jnp.int32) & jnp.int32(0x7FFFFFFF)
        m = lax.shift_right_logical(bits, 23) == b1
        key = lax.shift_right_logical(bits, 12) & jnp.int32(NB2 - 1)
        plsc.addupdate_scatter(cnt_ref, [key], ones, mask=m)
        return jnp.maximum(km, jnp.where(m, key, jnp.int32(0)))

    return jnp.max(kmax)


def _pass3(row_ref, cnt_ref, pre3):
    ones = jnp.full((L,), 1, jnp.int32)

    @plsc.parallel_loop(0, COLS // L, unroll=8)
    def _(j):
        v = row_ref[pl.ds(j * L, L)]
        bits = lax.bitcast_convert_type(v, jnp.int32) & jnp.int32(0x7FFFFFFF)
        m = lax.shift_right_logical(bits, 12) == pre3
        key = bits & jnp.int32(NB3 - 1)
        plsc.addupdate_scatter(cnt_ref, [key], ones, mask=m)


def _pass_sum(row_ref, thr_bits):
    zf = jnp.zeros((L,), jnp.float32)

    @plsc.parallel_loop(0, COLS // L, unroll=8, carry=zf)
    def acc(j, a):
        v = row_ref[pl.ds(j * L, L)]
        bits = lax.bitcast_convert_type(v, jnp.int32) & jnp.int32(0x7FFFFFFF)
        av = lax.bitcast_convert_type(bits, jnp.float32)
        return a + jnp.where(bits > thr_bits, av, zf)

    return jnp.sum(acc)


def _select_row(row_ref, h1c, h2c, h3c):
    bmax = _pass1(row_ref, h1c)
    kmax1 = lax.shift_right_logical(bmax, 23)
    b1, ac1 = _walk_lanes(h1c, kmax1, jnp.int32(TARGET))
    t2 = jnp.int32(TARGET) - ac1
    kmax2 = _pass2(row_ref, h2c, b1)
    b2, ac2, m2 = _walk(h2c, lax.shift_right_logical(kmax2, 4), t2)
    t3 = t2 - ac2
    pre3 = lax.shift_left(b1, 11) | b2
    need3 = jnp.logical_not((m2 == 1) & (t3 == 1))

    @pl.when(need3)
    def _():
        _pass3(row_ref, h3c, pre3)

    start3 = jnp.where(need3, jnp.int32(NB3 // L - 1), jnp.int32(-1))
    b3, _, _ = _walk(h3c, start3, t3)
    low = jnp.where(need3, b3, jnp.int32(NB3 - 1))
    thr_bits = lax.shift_left(pre3, 12) | low
    ans = _pass_sum(row_ref, thr_bits)
    _zero_cnt(h1c, (kmax1 + 1) * L)
    _zero_cnt(h2c, kmax2 + 1)
    _zero_cnt(h3c, jnp.where(need3, jnp.int32(NB3), jnp.int32(0)))
    return ans


def _make_selector():
    info = plsc.get_sparse_core_info()
    nw = info.num_cores * info.num_subcores
    rows_per_w = ROWS // nw
    mesh = plsc.VectorSubcoreMesh(core_axis_name="c", subcore_axis_name="s")

    @functools.partial(
        pl.kernel,
        mesh=mesh,
        compiler_params=pltpu.CompilerParams(needs_layout_passes=False),
        out_type=jax.ShapeDtypeStruct((nw, L), jnp.float32),
        scratch_types=[
            pltpu.VMEM((COLS,), jnp.float32),
            pltpu.VMEM((COLS,), jnp.float32),
            pltpu.VMEM((NB1 * L,), jnp.int32),
            pltpu.VMEM((NB2,), jnp.int32),
            pltpu.VMEM((NB3,), jnp.int32),
            pltpu.VMEM((L,), jnp.float32),
            pltpu.SemaphoreType.DMA,
            pltpu.SemaphoreType.DMA,
        ],
    )
    def sel(masks_hbm, out_hbm, row_a, row_b, h1c, h2c, h3c, stage,
            sem_a, sem_b):
        wid = lax.axis_index("s") * info.num_cores + lax.axis_index("c")
        row0 = wid * rows_per_w
        iota = lax.iota(jnp.int32, L)
        bufs = (row_a, row_b)
        sems = (sem_a, sem_b)

        handles = {}
        for r in range(2):
            handles[r] = pltpu.async_copy(masks_hbm.at[row0 + r], bufs[r],
                                          sems[r])
        _zero_cnt(h1c, NB1 * L)
        _zero_cnt(h2c, NB2)
        _zero_cnt(h3c, NB3)
        acc = jnp.zeros((L,), jnp.float32)
        for r in range(rows_per_w):
            handles[r].wait()
            ans = _select_row(bufs[r % 2], h1c, h2c, h3c)
            acc = jnp.where(iota == r, ans, acc)
            if r + 2 < rows_per_w:
                handles[r + 2] = pltpu.async_copy(
                    masks_hbm.at[row0 + r + 2], bufs[r % 2], sems[r % 2])
        stage[...] = acc
        pltpu.sync_copy(stage, out_hbm.at[wid])

    return sel


_selector = _make_selector()


def kernel(outputs_support, outputs_delete, targets, masks):
    parts = _selector(masks)
    return outputs_support[0] + 0.01 * jnp.sum(parts)
